# baseline (speedup 1.0000x reference)
import functools

import jax
import jax.numpy as jnp
from jax import lax
from jax.experimental import pallas as pl
from jax.experimental.pallas import tpu as pltpu

_R = 160
_C = 128
_NPAD = _R * _C
_K_PRE = 6000
_K_OUT = 1000
_THRESH = 0.7
_OUT_ROWS = 1024
_BIG = 2 ** 30


def _proposal_kernel(scores_ref, deltas_ref, anchors_ref, out_ref,
                     y1_ref, x1_ref, y2_ref, x2_ref, area_ref):
    nb = scores_ref.shape[0]
    ay1 = anchors_ref[0]
    ax1 = anchors_ref[1]
    ay2 = anchors_ref[2]
    ax2 = anchors_ref[3]
    a_h = ay2 - ay1
    a_w = ax2 - ax1

    flat = (lax.broadcasted_iota(jnp.int32, (_R, _C), 0) * _C
            + lax.broadcasted_iota(jnp.int32, (_R, _C), 1))
    lane = lax.broadcasted_iota(jnp.int32, (1, _C), 1)
    out_ref[...] = jnp.zeros_like(out_ref)

    cur_init = []
    for b in range(nb):
        s = scores_ref[b]
        dy = deltas_ref[b, 0] * 0.1
        dx = deltas_ref[b, 1] * 0.1
        dh = deltas_ref[b, 2] * 0.2
        dw = deltas_ref[b, 3] * 0.2

        cy = ay1 + 0.5 * a_h + dy * a_h
        cx = ax1 + 0.5 * a_w + dx * a_w
        h2 = a_h * jnp.exp(dh)
        w2 = a_w * jnp.exp(dw)
        y1 = cy - 0.5 * h2
        x1 = cx - 0.5 * w2
        y2 = y1 + h2
        x2 = x1 + w2
        y1 = jnp.clip(y1, 0.0, 1.0)
        x1 = jnp.clip(x1, 0.0, 1.0)
        y2 = jnp.clip(y2, 0.0, 1.0)
        x2 = jnp.clip(x2, 0.0, 1.0)
        y1_ref[b] = y1
        x1_ref[b] = x1
        y2_ref[b] = y2
        x2_ref[b] = x2
        area_ref[b] = (y2 - y1) * (x2 - x1)

        bits = lax.bitcast_convert_type(s, jnp.int32)

        def bs_val(_, lohi, bits=bits):
            lo, hi = lohi
            mid = lo + (hi - lo) // 2
            cnt = jnp.sum((bits >= mid).astype(jnp.int32))
            take = cnt >= _K_PRE
            return (jnp.where(take, mid, lo), jnp.where(take, hi, mid))

        lo, _hi = lax.fori_loop(0, 31, bs_val,
                                (jnp.int32(0), jnp.int32(0x40000000)))
        tau = lo
        n_gt = jnp.sum((bits > tau).astype(jnp.int32))
        m_ties = _K_PRE - n_gt
        eq = bits == tau

        def bs_idx(_, lohi, eq=eq, m_ties=m_ties):
            lo_i, hi_i = lohi
            mid = lo_i + (hi_i - lo_i) // 2
            cnt = jnp.sum((eq & (flat < mid)).astype(jnp.int32))
            take = cnt >= m_ties
            return (jnp.where(take, lo_i, mid), jnp.where(take, mid, hi_i))

        _lo_i, hi_i = lax.fori_loop(0, 15, bs_idx,
                                    (jnp.int32(0), jnp.int32(_NPAD)))
        include = (bits > tau) | (eq & (flat < hi_i))
        cur_init.append(jnp.where(include, s, -1.0))

    refs = (y1_ref, x1_ref, y2_ref, x2_ref, area_ref)
    idx_tiles = [lax.slice_in_dim(flat, 8 * t, 8 * (t + 1), axis=0)
                 for t in range(_R // 8)]

    def argmax_pair(cur):
        pairs = [(lax.slice_in_dim(cur, 8 * t, 8 * (t + 1), axis=0),
                  idx_tiles[t]) for t in range(_R // 8)]
        while len(pairs) > 1:
            nxt = []
            for j in range(0, len(pairs) - 1, 2):
                va, ia = pairs[j]
                vb, ib = pairs[j + 1]
                take = (vb > va) | ((vb == va) & (ib < ia))
                nxt.append((jnp.where(take, vb, va),
                            jnp.where(take, ib, ia)))
            if len(pairs) % 2:
                nxt.append(pairs[-1])
            pairs = nxt
        vf, if_ = pairs[0]
        mval = jnp.max(vf)
        fidx = jnp.min(jnp.where(vf == mval, if_, _BIG))
        return mval, fidx

    def cond(st):
        alive = False
        for b in range(nb):
            k, _, _, mval = st[b]
            alive = alive | ((k < _K_OUT) & (mval >= 0.0))
        return alive

    def body(st):
        new_st = []
        for b in range(nb):
            k, cur, fidx, mval = st[b]
            act = (k < _K_OUT) & (mval >= 0.0)
            fidx = jnp.where(act, fidx, 0)
            r = lax.shift_right_logical(fidx, 7)
            c = jnp.bitwise_and(fidx, _C - 1)
            hit = lane == c
            by1, bx1, by2, bx2, ba = [
                jnp.sum(jnp.where(hit, ref[b, pl.ds(r, 1), :], 0.0))
                for ref in refs]
            y1 = y1_ref[b]
            x1 = x1_ref[b]
            y2 = y2_ref[b]
            x2 = x2_ref[b]
            area = area_ref[b]
            yy1 = jnp.maximum(y1, by1)
            xx1 = jnp.maximum(x1, bx1)
            yy2 = jnp.minimum(y2, by2)
            xx2 = jnp.minimum(x2, bx2)
            inter = jnp.maximum(yy2 - yy1, 0.0) * jnp.maximum(xx2 - xx1, 0.0)
            union = area + ba - inter
            iou = jnp.where(union > 0.0, inter / union, 0.0)
            sup = (iou > _THRESH) | (flat == fidx)
            cur = jnp.where(act & sup, -1.0, cur)
            row = jnp.where(lane == 0, by1,
                            jnp.where(lane == 1, bx1,
                                      jnp.where(lane == 2, by2,
                                                jnp.where(lane == 3, bx2,
                                                          0.0))))

            krow = jnp.where(act, k, _K_OUT)
            out_ref[b, pl.ds(krow, 1), :] = row

            nmval, nfidx = argmax_pair(cur)
            new_st.append((k + act.astype(jnp.int32), cur, nfidx, nmval))
        return tuple(new_st)

    init = []
    for b in range(nb):
        mval0, fidx0 = argmax_pair(cur_init[b])
        init.append((jnp.int32(0), cur_init[b], fidx0, mval0))
    lax.while_loop(cond, body, tuple(init))


@functools.partial(jax.jit, static_argnames=())
def kernel(rpn_scores, rpn_bbox_delta, anchors):
    b = rpn_scores.shape[0]
    n = anchors.shape[0]
    pad = _NPAD - n
    scores = jnp.pad(rpn_scores[:, :, 1], ((0, 0), (0, pad)),
                     constant_values=-1.0).reshape(b, _R, _C)
    deltas = jnp.pad(jnp.transpose(rpn_bbox_delta, (0, 2, 1)),
                     ((0, 0), (0, 0), (0, pad))).reshape(b, 4, _R, _C)
    anch = jnp.pad(anchors.T, ((0, 0), (0, pad))).reshape(4, _R, _C)

    out = pl.pallas_call(
        _proposal_kernel,
        out_shape=jax.ShapeDtypeStruct((b, _OUT_ROWS, 128), jnp.float32),
        scratch_shapes=[pltpu.VMEM((b, _R, _C), jnp.float32)] * 5,
    )(scores, deltas, anch)
    return out[:, :_K_OUT, :4]


# speedup vs baseline: 1.1260x; 1.1260x over previous
import functools

import jax
import jax.numpy as jnp
from jax import lax
from jax.experimental import pallas as pl
from jax.experimental.pallas import tpu as pltpu

_R = 160
_C = 128
_NPAD = _R * _C
_K_PRE = 6000
_K_OUT = 1000
_THRESH = 0.7
_OUT_ROWS = 1024
_BIG = 2 ** 30


def _proposal_kernel(scores_ref, deltas_ref, anchors_ref, out_ref,
                     y1_ref, x1_ref, y2_ref, x2_ref, area_ref):
    nb = scores_ref.shape[0]
    ay1 = anchors_ref[0]
    ax1 = anchors_ref[1]
    ay2 = anchors_ref[2]
    ax2 = anchors_ref[3]
    a_h = ay2 - ay1
    a_w = ax2 - ax1

    flat = (lax.broadcasted_iota(jnp.int32, (_R, _C), 0) * _C
            + lax.broadcasted_iota(jnp.int32, (_R, _C), 1))
    lane = lax.broadcasted_iota(jnp.int32, (1, _C), 1)
    out_ref[...] = jnp.zeros_like(out_ref)

    cur_init = []
    for b in range(nb):
        s = scores_ref[b]
        dy = deltas_ref[b, 0] * 0.1
        dx = deltas_ref[b, 1] * 0.1
        dh = deltas_ref[b, 2] * 0.2
        dw = deltas_ref[b, 3] * 0.2

        cy = ay1 + 0.5 * a_h + dy * a_h
        cx = ax1 + 0.5 * a_w + dx * a_w
        h2 = a_h * jnp.exp(dh)
        w2 = a_w * jnp.exp(dw)
        y1 = cy - 0.5 * h2
        x1 = cx - 0.5 * w2
        y2 = y1 + h2
        x2 = x1 + w2
        y1 = jnp.clip(y1, 0.0, 1.0)
        x1 = jnp.clip(x1, 0.0, 1.0)
        y2 = jnp.clip(y2, 0.0, 1.0)
        x2 = jnp.clip(x2, 0.0, 1.0)
        y1_ref[b] = y1
        x1_ref[b] = x1
        y2_ref[b] = y2
        x2_ref[b] = x2
        area_ref[b] = (y2 - y1) * (x2 - x1)

        bits = lax.bitcast_convert_type(s, jnp.int32)

        def bs_val(_, lohi, bits=bits):
            lo, hi = lohi
            mid = lo + (hi - lo) // 2
            cnt = jnp.sum((bits >= mid).astype(jnp.int32))
            take = cnt >= _K_PRE
            return (jnp.where(take, mid, lo), jnp.where(take, hi, mid))

        lo, _hi = lax.fori_loop(0, 31, bs_val,
                                (jnp.int32(0), jnp.int32(0x40000000)))
        tau = lo
        n_gt = jnp.sum((bits > tau).astype(jnp.int32))
        m_ties = _K_PRE - n_gt
        eq = bits == tau

        def bs_idx(_, lohi, eq=eq, m_ties=m_ties):
            lo_i, hi_i = lohi
            mid = lo_i + (hi_i - lo_i) // 2
            cnt = jnp.sum((eq & (flat < mid)).astype(jnp.int32))
            take = cnt >= m_ties
            return (jnp.where(take, lo_i, mid), jnp.where(take, mid, hi_i))

        _lo_i, hi_i = lax.fori_loop(0, 15, bs_idx,
                                    (jnp.int32(0), jnp.int32(_NPAD)))
        include = (bits > tau) | (eq & (flat < hi_i))
        cur_init.append(jnp.where(include, s, -1.0))

    refs = (y1_ref, x1_ref, y2_ref, x2_ref, area_ref)

    def cond(st):
        alive = False
        for b in range(nb):
            k, _, mval = st[b]
            alive = alive | ((k < _K_OUT) & (mval >= 0.0))
        return alive

    def body(st):
        new_st = []
        for b in range(nb):
            k, cur, mval = st[b]
            act = (k < _K_OUT) & (mval >= 0.0)
            fidx = jnp.min(jnp.where(cur == mval, flat, _BIG))
            fidx = jnp.where(act, fidx, 0)
            r = lax.shift_right_logical(fidx, 7)
            c = jnp.bitwise_and(fidx, _C - 1)
            hit = lane == c
            by1, bx1, by2, bx2, ba = [
                jnp.sum(jnp.where(hit, ref[b, pl.ds(r, 1), :], 0.0))
                for ref in refs]
            y1 = y1_ref[b]
            x1 = x1_ref[b]
            y2 = y2_ref[b]
            x2 = x2_ref[b]
            area = area_ref[b]
            yy1 = jnp.maximum(y1, by1)
            xx1 = jnp.maximum(x1, bx1)
            yy2 = jnp.minimum(y2, by2)
            xx2 = jnp.minimum(x2, bx2)
            inter = jnp.maximum(yy2 - yy1, 0.0) * jnp.maximum(xx2 - xx1, 0.0)
            union = area + ba - inter
            iou = jnp.where(union > 0.0, inter / union, 0.0)
            sup = (iou > _THRESH) | (flat == fidx)
            cur = jnp.where(act & sup, -1.0, cur)
            row = jnp.where(lane == 0, by1,
                            jnp.where(lane == 1, bx1,
                                      jnp.where(lane == 2, by2,
                                                jnp.where(lane == 3, bx2,
                                                          0.0))))

            krow = jnp.where(act, k, _K_OUT)
            out_ref[b, pl.ds(krow, 1), :] = row

            new_st.append((k + act.astype(jnp.int32), cur, jnp.max(cur)))
        return tuple(new_st)

    init = tuple((jnp.int32(0), cur_init[b], jnp.max(cur_init[b]))
                 for b in range(nb))
    lax.while_loop(cond, body, init)


@functools.partial(jax.jit, static_argnames=())
def kernel(rpn_scores, rpn_bbox_delta, anchors):
    b = rpn_scores.shape[0]
    n = anchors.shape[0]
    pad = _NPAD - n
    scores = jnp.pad(rpn_scores[:, :, 1], ((0, 0), (0, pad)),
                     constant_values=-1.0).reshape(b, _R, _C)
    deltas = jnp.pad(jnp.transpose(rpn_bbox_delta, (0, 2, 1)),
                     ((0, 0), (0, 0), (0, pad))).reshape(b, 4, _R, _C)
    anch = jnp.pad(anchors.T, ((0, 0), (0, pad))).reshape(4, _R, _C)

    out = pl.pallas_call(
        _proposal_kernel,
        out_shape=jax.ShapeDtypeStruct((b, _OUT_ROWS, 128), jnp.float32),
        scratch_shapes=[pltpu.VMEM((b, _R, _C), jnp.float32)] * 5,
    )(scores, deltas, anch)
    return out[:, :_K_OUT, :4]


# speedup vs baseline: 1.1623x; 1.0322x over previous
import functools

import jax
import jax.numpy as jnp
from jax import lax
from jax.experimental import pallas as pl
from jax.experimental.pallas import tpu as pltpu

_R = 160
_C = 128
_NPAD = _R * _C
_K_PRE = 6000
_K_OUT = 1000
_THRESH = 0.7
_OUT_ROWS = 1024
_BIG = 2 ** 30


def _proposal_kernel(scores_ref, deltas_ref, anchors_ref, out_ref,
                     y1_ref, x1_ref, y2_ref, x2_ref, area_ref):
    nb = scores_ref.shape[0]
    ay1 = anchors_ref[0]
    ax1 = anchors_ref[1]
    ay2 = anchors_ref[2]
    ax2 = anchors_ref[3]
    a_h = ay2 - ay1
    a_w = ax2 - ax1

    flat = (lax.broadcasted_iota(jnp.int32, (_R, _C), 0) * _C
            + lax.broadcasted_iota(jnp.int32, (_R, _C), 1))
    lane = lax.broadcasted_iota(jnp.int32, (1, _C), 1)
    out_ref[...] = jnp.zeros_like(out_ref)

    cur_init = []
    for b in range(nb):
        s = scores_ref[b]
        dy = deltas_ref[b, 0] * 0.1
        dx = deltas_ref[b, 1] * 0.1
        dh = deltas_ref[b, 2] * 0.2
        dw = deltas_ref[b, 3] * 0.2

        cy = ay1 + 0.5 * a_h + dy * a_h
        cx = ax1 + 0.5 * a_w + dx * a_w
        h2 = a_h * jnp.exp(dh)
        w2 = a_w * jnp.exp(dw)
        y1 = cy - 0.5 * h2
        x1 = cx - 0.5 * w2
        y2 = y1 + h2
        x2 = x1 + w2
        y1 = jnp.clip(y1, 0.0, 1.0)
        x1 = jnp.clip(x1, 0.0, 1.0)
        y2 = jnp.clip(y2, 0.0, 1.0)
        x2 = jnp.clip(x2, 0.0, 1.0)
        y1_ref[b] = y1
        x1_ref[b] = x1
        y2_ref[b] = y2
        x2_ref[b] = x2
        area_ref[b] = (y2 - y1) * (x2 - x1)

        bits = lax.bitcast_convert_type(s, jnp.int32)

        def bs_val(_, lohi, bits=bits):
            lo, hi = lohi
            mid = lo + (hi - lo) // 2
            cnt = jnp.sum((bits >= mid).astype(jnp.int32))
            take = cnt >= _K_PRE
            return (jnp.where(take, mid, lo), jnp.where(take, hi, mid))

        lo, _hi = lax.fori_loop(0, 31, bs_val,
                                (jnp.int32(0), jnp.int32(0x40000000)))
        tau = lo
        n_gt = jnp.sum((bits > tau).astype(jnp.int32))
        m_ties = _K_PRE - n_gt
        eq = bits == tau

        def bs_idx(_, lohi, eq=eq, m_ties=m_ties):
            lo_i, hi_i = lohi
            mid = lo_i + (hi_i - lo_i) // 2
            cnt = jnp.sum((eq & (flat < mid)).astype(jnp.int32))
            take = cnt >= m_ties
            return (jnp.where(take, lo_i, mid), jnp.where(take, mid, hi_i))

        _lo_i, hi_i = lax.fori_loop(0, 15, bs_idx,
                                    (jnp.int32(0), jnp.int32(_NPAD)))
        include = (bits > tau) | (eq & (flat < hi_i))
        cur_init.append(jnp.where(include, s, -1.0))

    refs = (y1_ref, x1_ref, y2_ref, x2_ref, area_ref)

    def cond(st):
        alive = False
        for b in range(nb):
            k, _, mval = st[b]
            alive = alive | ((k < _K_OUT) & (mval >= 0.0))
        return alive

    def extract(b, fidx):
        r = lax.shift_right_logical(fidx, 7)
        c = jnp.bitwise_and(fidx, _C - 1)
        hit = lane == c
        return [jnp.sum(jnp.where(hit, ref[b, pl.ds(r, 1), :], 0.0))
                for ref in refs]

    def body(st):
        new_st = []
        for b in range(nb):
            k, cur, mval = st[b]
            act = (k < _K_OUT) & (mval >= 0.0)
            fidx = jnp.min(jnp.where(cur == mval, flat, _BIG))
            fidx = jnp.where(act, fidx, 0)
            cur_m = jnp.where(flat == fidx, -1.0, cur)
            mval2 = jnp.max(cur_m)
            fidx2 = jnp.min(jnp.where(cur_m == mval2, flat, _BIG))
            by1, bx1, by2, bx2, ba = extract(b, fidx)
            fidx2 = jnp.where(act & (mval2 >= 0.0), fidx2, 0)
            by1b, bx1b, by2b, bx2b, bab = extract(b, fidx2)
            pin = (jnp.maximum(jnp.minimum(by2, by2b)
                               - jnp.maximum(by1, by1b), 0.0)
                   * jnp.maximum(jnp.minimum(bx2, bx2b)
                                 - jnp.maximum(bx1, bx1b), 0.0))
            pun = bab + ba - pin
            piou = jnp.where(pun > 0.0, pin / pun, 0.0)
            act2 = (act & (mval2 >= 0.0) & (k + 1 < _K_OUT)
                    & jnp.logical_not(piou > _THRESH))
            y1 = y1_ref[b]
            x1 = x1_ref[b]
            y2 = y2_ref[b]
            x2 = x2_ref[b]
            area = area_ref[b]
            yy1 = jnp.maximum(y1, by1)
            xx1 = jnp.maximum(x1, bx1)
            yy2 = jnp.minimum(y2, by2)
            xx2 = jnp.minimum(x2, bx2)
            inter = jnp.maximum(yy2 - yy1, 0.0) * jnp.maximum(xx2 - xx1, 0.0)
            union = area + ba - inter
            iou = jnp.where(union > 0.0, inter / union, 0.0)
            sup1 = act & ((iou > _THRESH) | (flat == fidx))
            yy1b = jnp.maximum(y1, by1b)
            xx1b = jnp.maximum(x1, bx1b)
            yy2b = jnp.minimum(y2, by2b)
            xx2b = jnp.minimum(x2, bx2b)
            interb = (jnp.maximum(yy2b - yy1b, 0.0)
                      * jnp.maximum(xx2b - xx1b, 0.0))
            unionb = area + bab - interb
            ioub = jnp.where(unionb > 0.0, interb / unionb, 0.0)
            sup2 = act2 & ((ioub > _THRESH) | (flat == fidx2))
            cur = jnp.where(sup1 | sup2, -1.0, cur)
            row = jnp.where(lane == 0, by1,
                            jnp.where(lane == 1, bx1,
                                      jnp.where(lane == 2, by2,
                                                jnp.where(lane == 3, bx2,
                                                          0.0))))
            rowb = jnp.where(lane == 0, by1b,
                             jnp.where(lane == 1, bx1b,
                                       jnp.where(lane == 2, by2b,
                                                 jnp.where(lane == 3, bx2b,
                                                           0.0))))

            krow = jnp.where(act, k, _K_OUT)
            out_ref[b, pl.ds(krow, 1), :] = row
            krow2 = jnp.where(act2, k + 1, _K_OUT)
            out_ref[b, pl.ds(krow2, 1), :] = rowb

            k = k + act.astype(jnp.int32) + act2.astype(jnp.int32)
            new_st.append((k, cur, jnp.max(cur)))
        return tuple(new_st)

    init = tuple((jnp.int32(0), cur_init[b], jnp.max(cur_init[b]))
                 for b in range(nb))
    lax.while_loop(cond, body, init)


@functools.partial(jax.jit, static_argnames=())
def kernel(rpn_scores, rpn_bbox_delta, anchors):
    b = rpn_scores.shape[0]
    n = anchors.shape[0]
    pad = _NPAD - n
    scores = jnp.pad(rpn_scores[:, :, 1], ((0, 0), (0, pad)),
                     constant_values=-1.0).reshape(b, _R, _C)
    deltas = jnp.pad(jnp.transpose(rpn_bbox_delta, (0, 2, 1)),
                     ((0, 0), (0, 0), (0, pad))).reshape(b, 4, _R, _C)
    anch = jnp.pad(anchors.T, ((0, 0), (0, pad))).reshape(4, _R, _C)

    out = pl.pallas_call(
        _proposal_kernel,
        out_shape=jax.ShapeDtypeStruct((b, _OUT_ROWS, 128), jnp.float32),
        scratch_shapes=[pltpu.VMEM((b, _R, _C), jnp.float32)] * 5,
    )(scores, deltas, anch)
    return out[:, :_K_OUT, :4]
